# Initial kernel scaffold; baseline (speedup 1.0000x reference)
import jax
import jax.numpy as jnp
from jax.experimental import pallas as pl


def kernel(x, edge_index, edge_weight, h, c, params):
    raise NotImplementedError("write your pallas kernel here")


# speedup vs baseline: 24.0187x; 24.0187x over previous
import functools

import jax
import jax.numpy as jnp
from jax import lax
from jax.experimental import pallas as pl
from jax.experimental.pallas import tpu as pltpu
from jax.experimental.pallas import tpu_sc as plsc

N = 100000
NP = 100096
NSLICE = 6256
NSUB = 3128
E = 1600000
G = 128
EG = 12544
EP = EG * G
GP_DEG = EG // 32
GP_PROP = EG // 16
CH = 8
RB = 4000


def _sc_mesh():
    return plsc.VectorSubcoreMesh(
        core_axis_name="c", subcore_axis_name="s", num_cores=2, num_subcores=16
    )


def _deg_body(row_hbm, w_hbm, out_hbm, rowb, wb, stage, acc, sem):
    cid = lax.axis_index("c")
    sid = lax.axis_index("s")

    def zbody(i, carry):
        stage[pl.ds(i * 16, 16)] = jnp.zeros((16,), jnp.float32)
        return carry

    lax.fori_loop(0, NSLICE // 16, zbody, 0)
    pltpu.sync_copy(stage, acc.at[pl.ds(sid * NSLICE, NSLICE)])
    plsc.subcore_barrier()

    wkr = cid * 16 + sid

    def chunk(ci, carry):
        g0 = wkr * GP_DEG + ci * CH
        pltpu.sync_copy(row_hbm.at[pl.ds(g0, CH)], rowb)
        pltpu.sync_copy(w_hbm.at[pl.ds(g0, CH)], wb)
        cps = [
            pltpu.async_copy(wb.at[j], acc.at[rowb.at[j]], sem, add=True)
            for j in range(CH)
        ]
        for cp in cps:
            cp.wait()
        return carry

    lax.fori_loop(0, GP_DEG // CH, chunk, 0)
    plsc.subcore_barrier()
    pltpu.sync_copy(acc.at[pl.ds(sid * NSLICE, NSLICE)], stage)
    pltpu.sync_copy(stage, out_hbm.at[pl.ds(cid * NP + sid * NSLICE, NSLICE)])


def _deg_call(row2d, w2d):
    return pl.kernel(
        _deg_body,
        out_type=jax.ShapeDtypeStruct((2 * NP,), jnp.float32),
        mesh=_sc_mesh(),
        compiler_params=pltpu.CompilerParams(use_tc_tiling_on_sc=False),
        scratch_types=[
            pltpu.VMEM((CH, G), jnp.int32),
            pltpu.VMEM((CH, G), jnp.float32),
            pltpu.VMEM((NSLICE,), jnp.float32),
            pltpu.VMEM_SHARED((NP,), jnp.float32),
            pltpu.SemaphoreType.DMA,
        ],
    )(row2d, w2d)


def _prep_body(p0_ref, p1_ref, h_ref, dis_ref, hs_ref):
    deg = p0_ref[...] + p1_ref[...]
    dis = jnp.where(deg > 0.0, lax.rsqrt(deg), 0.0)
    dis_ref[...] = dis
    hsfull = -dis * h_ref[...]
    hs_ref[...] = jnp.stack([hsfull[:, :16], hsfull[:, 16:]])


def _prep_call(parts, h_pad):
    return pl.pallas_call(
        _prep_body,
        grid=(23,),
        in_specs=[
            pl.BlockSpec((4352, 1), lambda i: (i, 0)),
            pl.BlockSpec((4352, 1), lambda i: (23 + i, 0)),
            pl.BlockSpec((4352, 32), lambda i: (i, 0)),
        ],
        out_specs=[
            pl.BlockSpec((4352, 1), lambda i: (i, 0)),
            pl.BlockSpec((2, 4352, 16), lambda i: (0, i, 0)),
        ],
        out_shape=[
            jax.ShapeDtypeStruct((NP, 1), jnp.float32),
            jax.ShapeDtypeStruct((2, NP, 16), jnp.float32),
        ],
    )(parts, parts, h_pad)


def _prop_body(row_hbm, col_hbm, w_hbm, hs_hbm, out_hbm,
               rowb, colb, wb, rows, acc, sem):
    cid = lax.axis_index("c")
    sid = lax.axis_index("s")

    def zbody(i, carry):
        rows[i, :] = jnp.zeros((16,), jnp.float32)
        return carry

    lax.fori_loop(0, CH * G, zbody, 0)
    for k in range(6):
        pltpu.sync_copy(rows, acc.at[pl.ds(sid * NSLICE + k * (CH * G), CH * G)])
    pltpu.sync_copy(
        rows.at[pl.ds(0, NSLICE - 6 * CH * G)],
        acc.at[pl.ds(sid * NSLICE + 6 * CH * G, NSLICE - 6 * CH * G)],
    )
    plsc.subcore_barrier()

    roff = cid * NP

    def chunk(ci, carry):
        g0 = sid * GP_PROP + ci * CH
        pltpu.sync_copy(row_hbm.at[pl.ds(g0, CH)], rowb)
        pltpu.sync_copy(col_hbm.at[pl.ds(g0, CH)], colb)
        pltpu.sync_copy(w_hbm.at[pl.ds(g0, CH)], wb)

        def obody(i, carry2):
            j = i // CH
            l = i % CH
            rowb[j, pl.ds(l * 16, 16)] = rowb[j, pl.ds(l * 16, 16)] + roff
            return carry2

        lax.fori_loop(0, CH * (G // 16), obody, 0)
        cps = [
            pltpu.async_copy(
                hs_hbm.at[rowb.at[j]], rows.at[pl.ds(j * G, G)], sem
            )
            for j in range(CH)
        ]
        for cp in cps:
            cp.wait()

        def sbody(i, carry2):
            wv = wb[i // 8, pl.ds((i % 8) * 16, 16)]
            e0 = i * 16
            for u in range(16):
                rows[e0 + u, :] = rows[e0 + u, :] * wv[u]
            return carry2

        lax.fori_loop(0, (CH * G) // 16, sbody, 0)
        cps2 = [
            pltpu.async_copy(
                rows.at[pl.ds(j * G, G)], acc.at[colb.at[j]], sem, add=True
            )
            for j in range(CH)
        ]
        for cp in cps2:
            cp.wait()
        return carry

    lax.fori_loop(0, GP_PROP // CH, chunk, 0)
    plsc.subcore_barrier()
    pltpu.sync_copy(
        acc.at[pl.ds(sid * NSLICE, NSLICE)],
        out_hbm.at[pl.ds(cid * NP + sid * NSLICE, NSLICE)],
    )


def _prop_call(row2d, col2d, w2d, hs_flat):
    return pl.kernel(
        _prop_body,
        out_type=jax.ShapeDtypeStruct((2 * NP, 16), jnp.float32),
        mesh=_sc_mesh(),
        compiler_params=pltpu.CompilerParams(use_tc_tiling_on_sc=False),
        scratch_types=[
            pltpu.VMEM((CH, G), jnp.int32),
            pltpu.VMEM((CH, G), jnp.int32),
            pltpu.VMEM((CH, G), jnp.float32),
            pltpu.VMEM((CH * G, 16), jnp.float32),
            pltpu.VMEM_SHARED((NP, 16), jnp.float32),
            pltpu.SemaphoreType.DMA,
        ],
    )(row2d, col2d, w2d, hs_flat)


def _dense_body(x_ref, h_ref, c_ref, dis_ref, t0_ref, t1_ref,
                wcat_ref, w0_ref, w1a_ref, w1b_ref, bcat_ref,
                wci_ref, wcf_ref, wco_ref, lw_ref, lb_ref,
                y_ref, hh_ref, cc_ref):
    f32 = jnp.float32
    dis = dis_ref[...]
    s = (
        jnp.dot(x_ref[...], wcat_ref[...], preferred_element_type=f32)
        + jnp.dot(h_ref[...], w0_ref[...], preferred_element_type=f32)
        + jnp.dot(dis * t0_ref[...], w1a_ref[...], preferred_element_type=f32)
        + jnp.dot(dis * t1_ref[...], w1b_ref[...], preferred_element_type=f32)
        + bcat_ref[...]
    )
    cold = c_ref[...]
    gi = jax.nn.sigmoid(s[:, 0:32] + wci_ref[...] * cold)
    gf = jax.nn.sigmoid(s[:, 32:64] + wcf_ref[...] * cold)
    gt = jnp.tanh(s[:, 64:96])
    cnew = gf * cold + gi * gt
    go = jax.nn.sigmoid(s[:, 96:128] + wco_ref[...] * cnew)
    hnew = go * jnp.tanh(cnew)
    cc_ref[...] = cnew
    hh_ref[...] = hnew
    y_ref[...] = (
        jnp.dot(jax.nn.relu(hnew), lw_ref[...], preferred_element_type=f32)
        + lb_ref[...]
    )


def _dense_call(x, h, c, dis_n, t0, t1, wcat, w0, w1a, w1b, bcat,
                wci, wcf, wco, lw, lb):
    full = lambda shape: pl.BlockSpec(shape, lambda i: tuple(0 for _ in shape))
    return pl.pallas_call(
        _dense_body,
        grid=(N // RB,),
        in_specs=[
            pl.BlockSpec((RB, 8), lambda i: (i, 0)),
            pl.BlockSpec((RB, 32), lambda i: (i, 0)),
            pl.BlockSpec((RB, 32), lambda i: (i, 0)),
            pl.BlockSpec((RB, 1), lambda i: (i, 0)),
            pl.BlockSpec((RB, 16), lambda i: (i, 0)),
            pl.BlockSpec((RB, 16), lambda i: (i, 0)),
            full((8, 128)),
            full((32, 128)),
            full((16, 128)),
            full((16, 128)),
            full((1, 128)),
            full((1, 32)),
            full((1, 32)),
            full((1, 32)),
            full((32, 1)),
            full((1, 1)),
        ],
        out_specs=[
            pl.BlockSpec((RB, 1), lambda i: (i, 0)),
            pl.BlockSpec((RB, 32), lambda i: (i, 0)),
            pl.BlockSpec((RB, 32), lambda i: (i, 0)),
        ],
        out_shape=[
            jax.ShapeDtypeStruct((N, 1), jnp.float32),
            jax.ShapeDtypeStruct((N, 32), jnp.float32),
            jax.ShapeDtypeStruct((N, 32), jnp.float32),
        ],
    )(x, h, c, dis_n, t0, t1, wcat, w0, w1a, w1b, bcat, wci, wcf, wco, lw, lb)


def kernel(x, edge_index, edge_weight, h, c, params):
    p = params
    row = edge_index[0]
    col = edge_index[1]
    pad = EP - E
    row2d = jnp.pad(row, (0, pad)).reshape(EG, G)
    col2d = jnp.pad(col, (0, pad)).reshape(EG, G)
    w2d = jnp.pad(edge_weight, (0, pad)).reshape(EG, G)
    h_pad = jnp.pad(h, ((0, NP - N), (0, 0)))

    deg_parts = _deg_call(row2d, w2d)
    dis_p, hs = _prep_call(deg_parts.reshape(2 * NP, 1), h_pad)
    acc = _prop_call(row2d, col2d, w2d, hs.reshape(2 * NP, 16))

    dis_n = dis_p[:N]
    t0 = acc[:N]
    t1 = acc[NP:NP + N]

    gates = ["i", "f", "c", "o"]
    wcat = jnp.concatenate([p["W_" + g] for g in gates], axis=1)
    w0 = jnp.concatenate([p["conv_" + g + "_w"][0] for g in gates], axis=1)
    w1 = jnp.concatenate([p["conv_" + g + "_w"][1] for g in gates], axis=1)
    bcat = jnp.concatenate(
        [p["conv_" + g + "_b"][None, :] + p["b_" + g] for g in gates], axis=1
    )
    y, hn, cn = _dense_call(
        x, h, c, dis_n, t0, t1,
        wcat, w0, w1[:16], w1[16:], bcat,
        p["w_c_i"], p["w_c_f"], p["w_c_o"],
        p["lin_w"], p["lin_b"].reshape(1, 1),
    )
    return (y, hn, cn)


# speedup vs baseline: 24.7769x; 1.0316x over previous
import jax
import jax.numpy as jnp
from jax import lax
from jax.experimental import pallas as pl
from jax.experimental.pallas import tpu as pltpu
from jax.experimental.pallas import tpu_sc as plsc

N = 100000
NP = 100096
NSLICE = 6256
E = 1600000
G = 128
EG = E // G
CH = 8
RB = 4352

_SC_PARAMS = pltpu.CompilerParams(use_tc_tiling_on_sc=False)


def _sc_mesh():
    return plsc.VectorSubcoreMesh(
        core_axis_name="c", subcore_axis_name="s", num_cores=2, num_subcores=16
    )


def _deg_body(ei_hbm, w_hbm, out_hbm, rowb, wb, stage, acc, sem):
    cid = lax.axis_index("c")
    sid = lax.axis_index("s")

    def zbody(i, carry):
        stage[pl.ds(i * 16, 16)] = jnp.zeros((16,), jnp.float32)
        return carry

    lax.fori_loop(0, NSLICE // 16, zbody, 0)
    pltpu.sync_copy(stage, acc.at[pl.ds(sid * NSLICE, NSLICE)])
    plsc.subcore_barrier()

    wkr = cid * 16 + sid
    g0 = wkr * 390 + jnp.minimum(wkr, 20)
    rem = jnp.where(wkr < 20, 7, 6)

    def chunk(ci, carry):
        gr = g0 + ci * CH
        ld = [
            pltpu.async_copy(ei_hbm.at[0, pl.ds(gr, CH)], rowb, sem),
            pltpu.async_copy(w_hbm.at[pl.ds(gr, CH)], wb, sem),
        ]
        for cp in ld:
            cp.wait()
        cps = [
            pltpu.async_copy(wb.at[j], acc.at[rowb.at[j]], sem, add=True)
            for j in range(CH)
        ]
        for cp in cps:
            cp.wait()
        return carry

    lax.fori_loop(0, 48, chunk, 0)

    def tail(t, carry):
        gr = g0 + 384 + t
        ld = [
            pltpu.async_copy(ei_hbm.at[0, gr], rowb.at[0], sem),
            pltpu.async_copy(w_hbm.at[gr], wb.at[0], sem),
        ]
        for cp in ld:
            cp.wait()
        pltpu.async_copy(wb.at[0], acc.at[rowb.at[0]], sem, add=True).wait()
        return carry

    lax.fori_loop(0, rem, tail, 0)
    plsc.subcore_barrier()
    pltpu.sync_copy(acc.at[pl.ds(sid * NSLICE, NSLICE)], stage)
    pltpu.sync_copy(stage, out_hbm.at[pl.ds(cid * NP + sid * NSLICE, NSLICE)])


def _deg_call(ei3, w2):
    return pl.kernel(
        _deg_body,
        out_type=jax.ShapeDtypeStruct((2 * NP,), jnp.float32),
        mesh=_sc_mesh(),
        compiler_params=_SC_PARAMS,
        scratch_types=[
            pltpu.VMEM((CH, G), jnp.int32),
            pltpu.VMEM((CH, G), jnp.float32),
            pltpu.VMEM((NSLICE,), jnp.float32),
            pltpu.VMEM_SHARED((NP,), jnp.float32),
            pltpu.SemaphoreType.DMA,
        ],
    )(ei3, w2)


def _dis_body(p0_ref, p1_ref, dis_ref):
    deg = p0_ref[0] + p1_ref[0]
    dis_ref[...] = jnp.where(deg > 0.0, lax.rsqrt(deg), 0.0)


def _dis_call(parts3):
    return pl.pallas_call(
        _dis_body,
        grid=(1,),
        in_specs=[
            pl.BlockSpec((1, 782, 128), lambda i: (0, 0, 0)),
            pl.BlockSpec((1, 782, 128), lambda i: (1, 0, 0)),
        ],
        out_specs=pl.BlockSpec((782, 128), lambda i: (0, 0)),
        out_shape=jax.ShapeDtypeStruct((782, 128), jnp.float32),
    )(parts3, parts3)


def _scale_rows(rows, scb, nsub16, base, negate=False):
    def sbody(i, carry):
        sv = scb[pl.ds(i * 16, 16)]
        if negate:
            sv = -sv
        e0 = base + i * 16
        for u in range(16):
            rows[e0 + u, :] = rows[e0 + u, :] * sv[u]
        return carry

    lax.fori_loop(0, nsub16, sbody, 0)


def _prop_body(ei_hbm, w_hbm, h2_hbm, dis_hbm, out_hbm,
               rowb, colb, wb, disrb, scb, rows, disv, acc, sem):
    cid = lax.axis_index("c")
    sid = lax.axis_index("s")

    def zbody(i, carry):
        rows[i, :] = jnp.zeros((16,), jnp.float32)
        return carry

    lax.fori_loop(0, CH * G, zbody, 0)
    for k in range(6):
        pltpu.sync_copy(rows, acc.at[pl.ds(sid * NSLICE + k * (CH * G), CH * G)])
    pltpu.sync_copy(
        rows.at[pl.ds(0, NSLICE - 6 * CH * G)],
        acc.at[pl.ds(sid * NSLICE + 6 * CH * G, NSLICE - 6 * CH * G)],
    )
    plsc.subcore_barrier()

    g0 = sid * 781 + jnp.minimum(sid, 4)
    rem = jnp.where(sid < 4, 6, 5)

    def vloop(n, body):
        lax.fori_loop(0, n, lambda i, c: (body(i), c)[1], 0)

    def process(gr, nj):
        ld = [
            pltpu.async_copy(ei_hbm.at[0, pl.ds(gr, nj)], rowb.at[pl.ds(0, nj)], sem),
            pltpu.async_copy(ei_hbm.at[1, pl.ds(gr, nj)], colb.at[pl.ds(0, nj)], sem),
            pltpu.async_copy(w_hbm.at[pl.ds(gr, nj)], wb.at[pl.ds(0, nj)], sem),
        ]
        for cp in ld:
            cp.wait()
        gd = [
            pltpu.async_copy(dis_hbm.at[rowb.at[j]], disrb.at[j], sem)
            for j in range(nj)
        ]
        for cp in gd:
            cp.wait()
        def obody(i):
            j = i // (G // 16)
            l = i % (G // 16)
            sl = (j, pl.ds(l * 16, 16))
            rowb[sl] = rowb[sl] * 2 + cid

        vloop(nj * (G // 16), obody)
        gh = [
            pltpu.async_copy(
                h2_hbm.at[rowb.at[j]], rows.at[pl.ds(j * G, G)], sem
            )
            for j in range(nj)
        ]
        def mbody(i):
            j = i // (G // 16)
            l = i % (G // 16)
            scb[j, pl.ds(l * 16, 16)] = (
                wb[j, pl.ds(l * 16, 16)] * disrb[j, pl.ds(l * 16, 16)]
            )

        vloop(nj * (G // 16), mbody)
        for cp in gh:
            cp.wait()

        def sbody(i):
            j = i // (G // 16)
            l = i % (G // 16)
            sv = scb[j, pl.ds(l * 16, 16)]
            e0 = i * 16
            for u in range(16):
                rows[e0 + u, :] = rows[e0 + u, :] * sv[u]

        vloop(nj * (G // 16), sbody)
        sc = [
            pltpu.async_copy(
                rows.at[pl.ds(j * G, G)], acc.at[colb.at[j]], sem, add=True
            )
            for j in range(nj)
        ]
        for cp in sc:
            cp.wait()

    def chunk(ci, carry):
        process(g0 + ci * CH, CH)
        return carry

    lax.fori_loop(0, 97, chunk, 0)

    def tail(t, carry):
        process(g0 + 97 * CH + t, 1)
        return carry

    lax.fori_loop(0, rem, tail, 0)
    plsc.subcore_barrier()

    node0 = sid * NSLICE
    pltpu.sync_copy(dis_hbm.at[pl.ds(node0, NSLICE)], disv)
    for k in range(7):
        cnt = 1024 if k < 6 else NSLICE - 6 * 1024
        pltpu.sync_copy(
            acc.at[pl.ds(node0 + k * 1024, cnt)], rows.at[pl.ds(0, cnt)]
        )

        def wbody(i, carry, k=k, cnt=cnt):
            sv = -disv[pl.ds(k * 1024 + i * 16, 16)]
            e0 = i * 16
            for u in range(16):
                rows[e0 + u, :] = rows[e0 + u, :] * sv[u]
            return carry

        lax.fori_loop(0, cnt // 16, wbody, 0)
        pltpu.sync_copy(
            rows.at[pl.ds(0, cnt)],
            out_hbm.at[pl.ds(node0 + k * 1024, cnt), cid],
        )


def _prop_call(ei3, w2, h2, dis_flat):
    return pl.kernel(
        _prop_body,
        out_type=jax.ShapeDtypeStruct((NP, 2, 16), jnp.float32),
        mesh=_sc_mesh(),
        compiler_params=_SC_PARAMS,
        scratch_types=[
            pltpu.VMEM((CH, G), jnp.int32),
            pltpu.VMEM((CH, G), jnp.int32),
            pltpu.VMEM((CH, G), jnp.float32),
            pltpu.VMEM((CH, G), jnp.float32),
            pltpu.VMEM((CH, G), jnp.float32),
            pltpu.VMEM((CH * G, 16), jnp.float32),
            pltpu.VMEM((NSLICE,), jnp.float32),
            pltpu.VMEM_SHARED((NP, 16), jnp.float32),
            pltpu.SemaphoreType.DMA,
        ],
    )(ei3, w2, h2, dis_flat)


def _dense_body(x_ref, h_ref, c_ref, t_ref,
                wcat_ref, w0_ref, w1_ref, bcat_ref,
                wci_ref, wcf_ref, wco_ref, lw_ref, lb_ref,
                y_ref, hh_ref, cc_ref):
    f32 = jnp.float32
    s = (
        jnp.dot(x_ref[...], wcat_ref[...], preferred_element_type=f32)
        + jnp.dot(h_ref[...], w0_ref[...], preferred_element_type=f32)
        + jnp.dot(t_ref[...], w1_ref[...], preferred_element_type=f32)
        + bcat_ref[...]
    )
    cold = c_ref[...]
    gi = jax.nn.sigmoid(s[:, 0:32] + wci_ref[...] * cold)
    gf = jax.nn.sigmoid(s[:, 32:64] + wcf_ref[...] * cold)
    gt = jnp.tanh(s[:, 64:96])
    cnew = gf * cold + gi * gt
    go = jax.nn.sigmoid(s[:, 96:128] + wco_ref[...] * cnew)
    hnew = go * jnp.tanh(cnew)
    cc_ref[...] = cnew
    hh_ref[...] = hnew
    y_ref[...] = (
        jnp.dot(jax.nn.relu(hnew), lw_ref[...], preferred_element_type=f32)
        + lb_ref[...]
    )


def _dense_call(x, h, c, t, wcat, w0, w1, bcat, wci, wcf, wco, lw, lb):
    full = lambda shape: pl.BlockSpec(shape, lambda i: tuple(0 for _ in shape))
    nblk = (N + RB - 1) // RB
    return pl.pallas_call(
        _dense_body,
        grid=(nblk,),
        in_specs=[
            pl.BlockSpec((RB, 8), lambda i: (i, 0)),
            pl.BlockSpec((RB, 32), lambda i: (i, 0)),
            pl.BlockSpec((RB, 32), lambda i: (i, 0)),
            pl.BlockSpec((RB, 32), lambda i: (i, 0)),
            full((8, 128)),
            full((32, 128)),
            full((32, 128)),
            full((1, 128)),
            full((1, 32)),
            full((1, 32)),
            full((1, 32)),
            full((32, 1)),
            full((1, 1)),
        ],
        out_specs=[
            pl.BlockSpec((RB, 1), lambda i: (i, 0)),
            pl.BlockSpec((RB, 32), lambda i: (i, 0)),
            pl.BlockSpec((RB, 32), lambda i: (i, 0)),
        ],
        out_shape=[
            jax.ShapeDtypeStruct((N, 1), jnp.float32),
            jax.ShapeDtypeStruct((N, 32), jnp.float32),
            jax.ShapeDtypeStruct((N, 32), jnp.float32),
        ],
    )(x, h, c, t, wcat, w0, w1, bcat, wci, wcf, wco, lw, lb)


def kernel(x, edge_index, edge_weight, h, c, params):
    p = params
    ei3 = edge_index.reshape(2, EG, G)
    w2 = edge_weight.reshape(EG, G)
    h2 = h.reshape(2 * N, 16)

    deg_parts = _deg_call(ei3, w2)
    dis_g = _dis_call(deg_parts.reshape(2, 782, 128))
    dis_flat = dis_g.reshape(NP)
    t3 = _prop_call(ei3, w2, h2, dis_flat)
    t = t3.reshape(NP, 32)

    gates = ["i", "f", "c", "o"]
    wcat = jnp.concatenate([p["W_" + g] for g in gates], axis=1)
    w0 = jnp.concatenate([p["conv_" + g + "_w"][0] for g in gates], axis=1)
    w1 = jnp.concatenate([p["conv_" + g + "_w"][1] for g in gates], axis=1)
    bcat = jnp.concatenate(
        [p["conv_" + g + "_b"][None, :] + p["b_" + g] for g in gates], axis=1
    )
    y, hn, cn = _dense_call(
        x, h, c, t,
        wcat, w0, w1, bcat,
        p["w_c_i"], p["w_c_f"], p["w_c_o"],
        p["lin_w"], p["lin_b"].reshape(1, 1),
    )
    return (y, hn, cn)


# speedup vs baseline: 31.2525x; 1.2614x over previous
import jax
import jax.numpy as jnp
from jax import lax
from jax.experimental import pallas as pl
from jax.experimental.pallas import tpu as pltpu
from jax.experimental.pallas import tpu_sc as plsc

N = 100000
NP = 100096
NSLICE = 6256
E = 1600000
G = 128
EG = E // G
CH = 8
RB = 4352

_SC_PARAMS = pltpu.CompilerParams(use_tc_tiling_on_sc=False)


def _sc_mesh():
    return plsc.VectorSubcoreMesh(
        core_axis_name="c", subcore_axis_name="s", num_cores=2, num_subcores=16
    )


def _deg_body(ei_hbm, w_hbm, out_hbm, rowb, wb, stage, acc, sem):
    cid = lax.axis_index("c")
    sid = lax.axis_index("s")

    def zbody(i, carry):
        stage[pl.ds(i * 16, 16)] = jnp.zeros((16,), jnp.float32)
        return carry

    lax.fori_loop(0, NSLICE // 16, zbody, 0)
    pltpu.sync_copy(stage, acc.at[pl.ds(sid * NSLICE, NSLICE)])
    plsc.subcore_barrier()

    wkr = cid * 16 + sid
    g0 = wkr * 390 + jnp.minimum(wkr, 20)
    rem = jnp.where(wkr < 20, 7, 6)

    def chunk(ci, carry):
        gr = g0 + ci * CH
        ld = [
            pltpu.async_copy(ei_hbm.at[0, pl.ds(gr, CH)], rowb, sem),
            pltpu.async_copy(w_hbm.at[pl.ds(gr, CH)], wb, sem),
        ]
        for cp in ld:
            cp.wait()
        cps = [
            pltpu.async_copy(wb.at[j], acc.at[rowb.at[j]], sem, add=True)
            for j in range(CH)
        ]
        for cp in cps:
            cp.wait()
        return carry

    lax.fori_loop(0, 48, chunk, 0)

    def tail(t, carry):
        gr = g0 + 384 + t
        ld = [
            pltpu.async_copy(ei_hbm.at[0, gr], rowb.at[0], sem),
            pltpu.async_copy(w_hbm.at[gr], wb.at[0], sem),
        ]
        for cp in ld:
            cp.wait()
        pltpu.async_copy(wb.at[0], acc.at[rowb.at[0]], sem, add=True).wait()
        return carry

    lax.fori_loop(0, rem, tail, 0)
    plsc.subcore_barrier()
    pltpu.sync_copy(acc.at[pl.ds(sid * NSLICE, NSLICE)], stage)
    pltpu.sync_copy(stage, out_hbm.at[pl.ds(cid * NP + sid * NSLICE, NSLICE)])


def _deg_call(ei3, w2):
    return pl.kernel(
        _deg_body,
        out_type=jax.ShapeDtypeStruct((2 * NP,), jnp.float32),
        mesh=_sc_mesh(),
        compiler_params=_SC_PARAMS,
        scratch_types=[
            pltpu.VMEM((CH, G), jnp.int32),
            pltpu.VMEM((CH, G), jnp.float32),
            pltpu.VMEM((NSLICE,), jnp.float32),
            pltpu.VMEM_SHARED((NP,), jnp.float32),
            pltpu.SemaphoreType.DMA,
        ],
    )(ei3, w2)


def _dis_body(p0_ref, p1_ref, dis_ref):
    deg = p0_ref[0] + p1_ref[0]
    dis_ref[...] = jnp.where(deg > 0.0, lax.rsqrt(deg), 0.0)


def _dis_call(parts3):
    return pl.pallas_call(
        _dis_body,
        grid=(1,),
        in_specs=[
            pl.BlockSpec((1, 782, 128), lambda i: (0, 0, 0)),
            pl.BlockSpec((1, 782, 128), lambda i: (1, 0, 0)),
        ],
        out_specs=pl.BlockSpec((782, 128), lambda i: (0, 0)),
        out_shape=jax.ShapeDtypeStruct((782, 128), jnp.float32),
    )(parts3, parts3)


def _scale_rows(rows, scb, nsub16, base, negate=False):
    def sbody(i, carry):
        sv = scb[pl.ds(i * 16, 16)]
        if negate:
            sv = -sv
        e0 = base + i * 16
        for u in range(16):
            rows[e0 + u, :] = rows[e0 + u, :] * sv[u]
        return carry

    lax.fori_loop(0, nsub16, sbody, 0)


def _prop_body(ei_hbm, w_hbm, h2_hbm, dis_hbm, out_hbm,
               rowb, colb, wb, disrb, scb, rows, disv, acc, sem):
    cid = lax.axis_index("c")
    sid = lax.axis_index("s")

    def zbody(i, carry):
        rows[i, :] = jnp.zeros((16,), jnp.float32)
        return carry

    lax.fori_loop(0, CH * G, zbody, 0)
    for k in range(6):
        pltpu.sync_copy(rows, acc.at[pl.ds(sid * NSLICE + k * (CH * G), CH * G)])
    pltpu.sync_copy(
        rows.at[pl.ds(0, NSLICE - 6 * CH * G)],
        acc.at[pl.ds(sid * NSLICE + 6 * CH * G, NSLICE - 6 * CH * G)],
    )
    plsc.subcore_barrier()

    g0 = sid * 781 + jnp.minimum(sid, 4)
    rem = jnp.where(sid < 4, 6, 5)

    def vloop(n, body):
        lax.fori_loop(0, n, lambda i, c: (body(i), c)[1], 0)

    def process(gr, nj):
        ld = [
            pltpu.async_copy(ei_hbm.at[0, pl.ds(gr, nj)], rowb.at[pl.ds(0, nj)], sem),
            pltpu.async_copy(ei_hbm.at[1, pl.ds(gr, nj)], colb.at[pl.ds(0, nj)], sem),
            pltpu.async_copy(w_hbm.at[pl.ds(gr, nj)], wb.at[pl.ds(0, nj)], sem),
        ]
        for cp in ld:
            cp.wait()
        gd = [
            pltpu.async_copy(dis_hbm.at[rowb.at[j]], disrb.at[j], sem)
            for j in range(nj)
        ]
        for cp in gd:
            cp.wait()
        def obody(i):
            j = i // (G // 16)
            l = i % (G // 16)
            sl = (j, pl.ds(l * 16, 16))
            rowb[sl] = rowb[sl] * 2 + cid

        vloop(nj * (G // 16), obody)
        gh = [
            pltpu.async_copy(
                h2_hbm.at[rowb.at[j]], rows.at[pl.ds(j * G, G)], sem
            )
            for j in range(nj)
        ]
        def mbody(i):
            j = i // (G // 16)
            l = i % (G // 16)
            scb[j, pl.ds(l * 16, 16)] = (
                wb[j, pl.ds(l * 16, 16)] * disrb[j, pl.ds(l * 16, 16)]
            )

        vloop(nj * (G // 16), mbody)
        for cp in gh:
            cp.wait()

        def sbody(i):
            j = i // (G // 16)
            l = i % (G // 16)
            sv = scb[j, pl.ds(l * 16, 16)]
            e0 = i * 16
            for u in range(16):
                rows[e0 + u, :] = rows[e0 + u, :] * sv[u]

        vloop(nj * (G // 16), sbody)
        sc = [
            pltpu.async_copy(
                rows.at[pl.ds(j * G, G)], acc.at[colb.at[j]], sem, add=True
            )
            for j in range(nj)
        ]
        for cp in sc:
            cp.wait()

    def chunk(ci, carry):
        process(g0 + ci * CH, CH)
        return carry

    lax.fori_loop(0, 97, chunk, 0)

    def tail(t, carry):
        process(g0 + 97 * CH + t, 1)
        return carry

    lax.fori_loop(0, rem, tail, 0)
    plsc.subcore_barrier()

    node0 = sid * NSLICE
    pltpu.sync_copy(dis_hbm.at[pl.ds(node0, NSLICE)], disv)
    for k in range(7):
        cnt = 1024 if k < 6 else NSLICE - 6 * 1024
        pltpu.sync_copy(
            acc.at[pl.ds(node0 + k * 1024, cnt)], rows.at[pl.ds(0, cnt)]
        )

        def wbody(i, carry, k=k, cnt=cnt):
            sv = -disv[pl.ds(k * 1024 + i * 16, 16)]
            e0 = i * 16
            for u in range(16):
                rows[e0 + u, :] = rows[e0 + u, :] * sv[u]
            return carry

        lax.fori_loop(0, cnt // 16, wbody, 0)
        pltpu.sync_copy(
            rows.at[pl.ds(0, cnt)],
            out_hbm.at[pl.ds(cid * NP + node0 + k * 1024, cnt)],
        )


def _prop_call(ei3, w2, h2, dis_flat):
    return pl.kernel(
        _prop_body,
        out_type=jax.ShapeDtypeStruct((2 * NP, 16), jnp.float32),
        mesh=_sc_mesh(),
        compiler_params=_SC_PARAMS,
        scratch_types=[
            pltpu.VMEM((CH, G), jnp.int32),
            pltpu.VMEM((CH, G), jnp.int32),
            pltpu.VMEM((CH, G), jnp.float32),
            pltpu.VMEM((CH, G), jnp.float32),
            pltpu.VMEM((CH, G), jnp.float32),
            pltpu.VMEM((CH * G, 16), jnp.float32),
            pltpu.VMEM((NSLICE,), jnp.float32),
            pltpu.VMEM_SHARED((NP, 16), jnp.float32),
            pltpu.SemaphoreType.DMA,
        ],
    )(ei3, w2, h2, dis_flat)


def _dense_body(x_ref, h_ref, c_ref, t0_ref, t1_ref,
                wcat_ref, w0_ref, w1a_ref, w1b_ref, bcat_ref,
                wci_ref, wcf_ref, wco_ref, lw_ref, lb_ref,
                y_ref, hh_ref, cc_ref):
    f32 = jnp.float32
    s = (
        jnp.dot(x_ref[...], wcat_ref[...], preferred_element_type=f32)
        + jnp.dot(h_ref[...], w0_ref[...], preferred_element_type=f32)
        + jnp.dot(t0_ref[...], w1a_ref[...], preferred_element_type=f32)
        + jnp.dot(t1_ref[...], w1b_ref[...], preferred_element_type=f32)
        + bcat_ref[...]
    )
    cold = c_ref[...]
    gi = jax.nn.sigmoid(s[:, 0:32] + wci_ref[...] * cold)
    gf = jax.nn.sigmoid(s[:, 32:64] + wcf_ref[...] * cold)
    gt = jnp.tanh(s[:, 64:96])
    cnew = gf * cold + gi * gt
    go = jax.nn.sigmoid(s[:, 96:128] + wco_ref[...] * cnew)
    hnew = go * jnp.tanh(cnew)
    cc_ref[...] = cnew
    hh_ref[...] = hnew
    y_ref[...] = (
        jnp.dot(jax.nn.relu(hnew), lw_ref[...], preferred_element_type=f32)
        + lb_ref[...]
    )


def _dense_call(x, h, c, t2, wcat, w0, w1a, w1b, bcat, wci, wcf, wco, lw, lb):
    full = lambda shape: pl.BlockSpec(shape, lambda i: tuple(0 for _ in shape))
    nblk = (N + RB - 1) // RB
    return pl.pallas_call(
        _dense_body,
        grid=(nblk,),
        in_specs=[
            pl.BlockSpec((RB, 8), lambda i: (i, 0)),
            pl.BlockSpec((RB, 32), lambda i: (i, 0)),
            pl.BlockSpec((RB, 32), lambda i: (i, 0)),
            pl.BlockSpec((RB, 16), lambda i: (i, 0)),
            pl.BlockSpec((RB, 16), lambda i: (23 + i, 0)),
            full((8, 128)),
            full((32, 128)),
            full((16, 128)),
            full((16, 128)),
            full((1, 128)),
            full((1, 32)),
            full((1, 32)),
            full((1, 32)),
            full((32, 1)),
            full((1, 1)),
        ],
        out_specs=[
            pl.BlockSpec((RB, 1), lambda i: (i, 0)),
            pl.BlockSpec((RB, 32), lambda i: (i, 0)),
            pl.BlockSpec((RB, 32), lambda i: (i, 0)),
        ],
        out_shape=[
            jax.ShapeDtypeStruct((N, 1), jnp.float32),
            jax.ShapeDtypeStruct((N, 32), jnp.float32),
            jax.ShapeDtypeStruct((N, 32), jnp.float32),
        ],
    )(x, h, c, t2, t2, wcat, w0, w1a, w1b, bcat, wci, wcf, wco, lw, lb)


def kernel(x, edge_index, edge_weight, h, c, params):
    p = params
    ei3 = edge_index.reshape(2, EG, G)
    w2 = edge_weight.reshape(EG, G)
    h2 = h.reshape(2 * N, 16)

    deg_parts = _deg_call(ei3, w2)
    dis_g = _dis_call(deg_parts.reshape(2, 782, 128))
    dis_flat = dis_g.reshape(NP)
    t2 = _prop_call(ei3, w2, h2, dis_flat)

    gates = ["i", "f", "c", "o"]
    wcat = jnp.concatenate([p["W_" + g] for g in gates], axis=1)
    w0 = jnp.concatenate([p["conv_" + g + "_w"][0] for g in gates], axis=1)
    w1 = jnp.concatenate([p["conv_" + g + "_w"][1] for g in gates], axis=1)
    bcat = jnp.concatenate(
        [p["conv_" + g + "_b"][None, :] + p["b_" + g] for g in gates], axis=1
    )
    y, hn, cn = _dense_call(
        x, h, c, t2,
        wcat, w0, w1[:16], w1[16:], bcat,
        p["w_c_i"], p["w_c_f"], p["w_c_o"],
        p["lin_w"], p["lin_b"].reshape(1, 1),
    )
    return (y, hn, cn)


# speedup vs baseline: 34.9571x; 1.1185x over previous
import jax
import jax.numpy as jnp
from jax import lax
from jax.experimental import pallas as pl
from jax.experimental.pallas import tpu as pltpu
from jax.experimental.pallas import tpu_sc as plsc

N = 100000
NP = 100096
NSLICE = 6256
E = 1600000
G = 128
EG = E // G
CH = 8
PC = 4
RB = 4352

_SC_PARAMS = pltpu.CompilerParams(use_tc_tiling_on_sc=False)


def _sc_mesh():
    return plsc.VectorSubcoreMesh(
        core_axis_name="c", subcore_axis_name="s", num_cores=2, num_subcores=16
    )


def _deg_body(ei_hbm, w_hbm, out_hbm, rowb, wb, stage, acc, sem):
    cid = lax.axis_index("c")
    sid = lax.axis_index("s")

    def zbody(i, carry):
        stage[pl.ds(i * 16, 16)] = jnp.zeros((16,), jnp.float32)
        return carry

    lax.fori_loop(0, NSLICE // 16, zbody, 0)
    pltpu.sync_copy(stage, acc.at[pl.ds(sid * NSLICE, NSLICE)])
    plsc.subcore_barrier()

    wkr = cid * 16 + sid
    g0 = wkr * 390 + jnp.minimum(wkr, 20)
    rem = jnp.where(wkr < 20, 7, 6)

    def chunk(ci, carry):
        gr = g0 + ci * CH
        ld = [
            pltpu.async_copy(ei_hbm.at[0, pl.ds(gr, CH)], rowb, sem),
            pltpu.async_copy(w_hbm.at[pl.ds(gr, CH)], wb, sem),
        ]
        for cp in ld:
            cp.wait()
        cps = [
            pltpu.async_copy(wb.at[j], acc.at[rowb.at[j]], sem, add=True)
            for j in range(CH)
        ]
        for cp in cps:
            cp.wait()
        return carry

    lax.fori_loop(0, 48, chunk, 0)

    def tail(t, carry):
        gr = g0 + 384 + t
        ld = [
            pltpu.async_copy(ei_hbm.at[0, gr], rowb.at[0], sem),
            pltpu.async_copy(w_hbm.at[gr], wb.at[0], sem),
        ]
        for cp in ld:
            cp.wait()
        pltpu.async_copy(wb.at[0], acc.at[rowb.at[0]], sem, add=True).wait()
        return carry

    lax.fori_loop(0, rem, tail, 0)
    plsc.subcore_barrier()
    pltpu.sync_copy(acc.at[pl.ds(sid * NSLICE, NSLICE)], stage)
    pltpu.sync_copy(stage, out_hbm.at[pl.ds(cid * NP + sid * NSLICE, NSLICE)])


def _deg_call(ei3, w2):
    return pl.kernel(
        _deg_body,
        out_type=jax.ShapeDtypeStruct((2 * NP,), jnp.float32),
        mesh=_sc_mesh(),
        compiler_params=_SC_PARAMS,
        scratch_types=[
            pltpu.VMEM((CH, G), jnp.int32),
            pltpu.VMEM((CH, G), jnp.float32),
            pltpu.VMEM((NSLICE,), jnp.float32),
            pltpu.VMEM_SHARED((NP,), jnp.float32),
            pltpu.SemaphoreType.DMA,
        ],
    )(ei3, w2)


def _dis_body(p0_ref, p1_ref, dis_ref):
    deg = p0_ref[0] + p1_ref[0]
    dis_ref[...] = jnp.where(deg > 0.0, lax.rsqrt(deg), 0.0)


def _dis_call(parts3):
    return pl.pallas_call(
        _dis_body,
        grid=(1,),
        in_specs=[
            pl.BlockSpec((1, 782, 128), lambda i: (0, 0, 0)),
            pl.BlockSpec((1, 782, 128), lambda i: (1, 0, 0)),
        ],
        out_specs=pl.BlockSpec((782, 128), lambda i: (0, 0)),
        out_shape=jax.ShapeDtypeStruct((782, 128), jnp.float32),
    )(parts3, parts3)


def _scale_rows(rows, scb, nsub16, base, negate=False):
    def sbody(i, carry):
        sv = scb[pl.ds(i * 16, 16)]
        if negate:
            sv = -sv
        e0 = base + i * 16
        for u in range(16):
            rows[e0 + u, :] = rows[e0 + u, :] * sv[u]
        return carry

    lax.fori_loop(0, nsub16, sbody, 0)


def _prop_body(ei_hbm, w_hbm, h2_hbm, dis_hbm, out_hbm,
               rowb0, rowb1, colb0, colb1, csb0, csb1, wbf, disrb,
               scb0, scb1, rows0, rows1, disv, acc,
               semi, semg0, semg1, sems0, sems1):
    cid = lax.axis_index("c")
    sid = lax.axis_index("s")
    rowb = (rowb0, rowb1)
    colb = (colb0, colb1)
    csb = (csb0, csb1)
    scb = (scb0, scb1)
    rows = (rows0, rows1)
    semg = (semg0, semg1)
    sems = (sems0, sems1)

    def zbody(i, carry):
        rows0[i, :] = jnp.zeros((16,), jnp.float32)
        return carry

    lax.fori_loop(0, PC * G, zbody, 0)
    nz = PC * G
    for k in range(NSLICE // nz):
        pltpu.sync_copy(rows0, acc.at[pl.ds(sid * NSLICE + k * nz, nz)])
    rz = NSLICE - (NSLICE // nz) * nz
    pltpu.sync_copy(
        rows0.at[pl.ds(0, rz)],
        acc.at[pl.ds(sid * NSLICE + (NSLICE // nz) * nz, rz)],
    )
    plsc.subcore_barrier()

    g0 = sid * 781 + jnp.minimum(sid, 4)
    rem = jnp.where(sid < 4, 2, 1)

    def vloop(n, body):
        lax.fori_loop(0, n, lambda i, c: (body(i), c)[1], 0)

    def drain(sem_, buf):
        pltpu.make_async_copy(h2_hbm.at[pl.ds(0, PC * G)], buf, sem_).wait()

    def stage_a(ci, s, drain_sc):
        gr = g0 + ci * PC
        ld = [
            pltpu.async_copy(ei_hbm.at[0, pl.ds(gr, PC)], rowb[s], semi),
            pltpu.async_copy(ei_hbm.at[1, pl.ds(gr, PC)], colb[s], semi),
            pltpu.async_copy(w_hbm.at[pl.ds(gr, PC)], wbf, semi),
        ]
        for cp in ld:
            cp.wait()
        gd = [
            pltpu.async_copy(dis_hbm.at[rowb[s].at[j]], disrb.at[j], semi)
            for j in range(PC)
        ]
        for cp in gd:
            cp.wait()
        if drain_sc:
            drain(sems[s], rows[s])

        def obody(i):
            j = i // (G // 16)
            l = i % (G // 16)
            sl = (j, pl.ds(l * 16, 16))
            rowb[s][sl] = rowb[s][sl] * 2 + cid
            scb[s][sl] = wbf[sl] * disrb[sl]
            csb[s][sl] = colb[s][sl]

        vloop(PC * (G // 16), obody)
        for j in range(PC):
            pltpu.async_copy(
                h2_hbm.at[rowb[s].at[j]], rows[s].at[pl.ds(j * G, G)], semg[s]
            )

    def stage_b(s):
        drain(semg[s], rows[s])

        def sbody(i):
            j = i // (G // 16)
            l = i % (G // 16)
            sv = scb[s][j, pl.ds(l * 16, 16)]
            e0 = i * 16
            for u in range(16):
                rows[s][e0 + u, :] = rows[s][e0 + u, :] * sv[u]

        vloop(PC * (G // 16), sbody)
        for j in range(PC):
            pltpu.async_copy(
                rows[s].at[pl.ds(j * G, G)], acc.at[csb[s].at[j]],
                sems[s], add=True,
            )

    stage_a(0, 0, False)
    stage_a(1, 1, False)
    stage_b(0)

    def ring(k, carry):
        ci = 2 * k
        stage_a(ci, 0, True)
        stage_b(1)
        stage_a(ci + 1, 1, True)
        stage_b(0)
        return carry

    lax.fori_loop(1, 97, ring, 0)
    stage_a(194, 0, True)
    stage_b(1)
    stage_b(0)
    drain(sems[0], rows[0])
    drain(sems[1], rows[1])

    def tail(t, carry):
        gr = g0 + 780 + t
        ld = [
            pltpu.async_copy(ei_hbm.at[0, gr], rowb0.at[0], semi),
            pltpu.async_copy(ei_hbm.at[1, gr], colb0.at[0], semi),
            pltpu.async_copy(w_hbm.at[gr], wbf.at[0], semi),
        ]
        for cp in ld:
            cp.wait()
        pltpu.async_copy(dis_hbm.at[rowb0.at[0]], disrb.at[0], semi).wait()

        def obody(i):
            sl = (0, pl.ds(i * 16, 16))
            rowb0[sl] = rowb0[sl] * 2 + cid
            scb0[sl] = wbf[sl] * disrb[sl]

        vloop(G // 16, obody)
        pltpu.async_copy(
            h2_hbm.at[rowb0.at[0]], rows0.at[pl.ds(0, G)], semi
        ).wait()

        def sbody(i):
            sv = scb0[0, pl.ds(i * 16, 16)]
            e0 = i * 16
            for u in range(16):
                rows0[e0 + u, :] = rows0[e0 + u, :] * sv[u]

        vloop(G // 16, sbody)
        pltpu.async_copy(
            rows0.at[pl.ds(0, G)], acc.at[colb0.at[0]], semi, add=True
        ).wait()
        return carry

    lax.fori_loop(0, rem, tail, 0)
    plsc.subcore_barrier()

    node0 = sid * NSLICE
    pltpu.sync_copy(dis_hbm.at[pl.ds(node0, NSLICE)], disv)
    nwb = PC * G
    for k in range(13):
        cnt = nwb if k < 12 else NSLICE - 12 * nwb
        pltpu.sync_copy(
            acc.at[pl.ds(node0 + k * nwb, cnt)], rows0.at[pl.ds(0, cnt)]
        )

        def wbody(i, carry, k=k):
            sv = -disv[pl.ds(k * nwb + i * 16, 16)]
            e0 = i * 16
            for u in range(16):
                rows0[e0 + u, :] = rows0[e0 + u, :] * sv[u]
            return carry

        lax.fori_loop(0, cnt // 16, wbody, 0)
        pltpu.sync_copy(
            rows0.at[pl.ds(0, cnt)],
            out_hbm.at[pl.ds(cid * NP + node0 + k * nwb, cnt)],
        )


def _prop_call(ei3, w2, h2, dis_flat):
    return pl.kernel(
        _prop_body,
        out_type=jax.ShapeDtypeStruct((2 * NP, 16), jnp.float32),
        mesh=_sc_mesh(),
        compiler_params=_SC_PARAMS,
        scratch_types=[
            pltpu.VMEM((PC, G), jnp.int32),
            pltpu.VMEM((PC, G), jnp.int32),
            pltpu.VMEM((PC, G), jnp.int32),
            pltpu.VMEM((PC, G), jnp.int32),
            pltpu.VMEM((PC, G), jnp.int32),
            pltpu.VMEM((PC, G), jnp.int32),
            pltpu.VMEM((PC, G), jnp.float32),
            pltpu.VMEM((PC, G), jnp.float32),
            pltpu.VMEM((PC, G), jnp.float32),
            pltpu.VMEM((PC, G), jnp.float32),
            pltpu.VMEM((PC * G, 16), jnp.float32),
            pltpu.VMEM((PC * G, 16), jnp.float32),
            pltpu.VMEM((NSLICE,), jnp.float32),
            pltpu.VMEM_SHARED((NP, 16), jnp.float32),
            pltpu.SemaphoreType.DMA,
            pltpu.SemaphoreType.DMA,
            pltpu.SemaphoreType.DMA,
            pltpu.SemaphoreType.DMA,
            pltpu.SemaphoreType.DMA,
        ],
    )(ei3, w2, h2, dis_flat)


def _dense_body(x_ref, h_ref, c_ref, t0_ref, t1_ref,
                wcat_ref, w0_ref, w1a_ref, w1b_ref, bcat_ref,
                wci_ref, wcf_ref, wco_ref, lw_ref, lb_ref,
                y_ref, hh_ref, cc_ref):
    f32 = jnp.float32
    s = (
        jnp.dot(x_ref[...], wcat_ref[...], preferred_element_type=f32)
        + jnp.dot(h_ref[...], w0_ref[...], preferred_element_type=f32)
        + jnp.dot(t0_ref[...], w1a_ref[...], preferred_element_type=f32)
        + jnp.dot(t1_ref[...], w1b_ref[...], preferred_element_type=f32)
        + bcat_ref[...]
    )
    cold = c_ref[...]
    gi = jax.nn.sigmoid(s[:, 0:32] + wci_ref[...] * cold)
    gf = jax.nn.sigmoid(s[:, 32:64] + wcf_ref[...] * cold)
    gt = jnp.tanh(s[:, 64:96])
    cnew = gf * cold + gi * gt
    go = jax.nn.sigmoid(s[:, 96:128] + wco_ref[...] * cnew)
    hnew = go * jnp.tanh(cnew)
    cc_ref[...] = cnew
    hh_ref[...] = hnew
    y_ref[...] = (
        jnp.dot(jax.nn.relu(hnew), lw_ref[...], preferred_element_type=f32)
        + lb_ref[...]
    )


def _dense_call(x, h, c, t2, wcat, w0, w1a, w1b, bcat, wci, wcf, wco, lw, lb):
    full = lambda shape: pl.BlockSpec(shape, lambda i: tuple(0 for _ in shape))
    nblk = (N + RB - 1) // RB
    return pl.pallas_call(
        _dense_body,
        grid=(nblk,),
        in_specs=[
            pl.BlockSpec((RB, 8), lambda i: (i, 0)),
            pl.BlockSpec((RB, 32), lambda i: (i, 0)),
            pl.BlockSpec((RB, 32), lambda i: (i, 0)),
            pl.BlockSpec((RB, 16), lambda i: (i, 0)),
            pl.BlockSpec((RB, 16), lambda i: (23 + i, 0)),
            full((8, 128)),
            full((32, 128)),
            full((16, 128)),
            full((16, 128)),
            full((1, 128)),
            full((1, 32)),
            full((1, 32)),
            full((1, 32)),
            full((32, 1)),
            full((1, 1)),
        ],
        out_specs=[
            pl.BlockSpec((RB, 1), lambda i: (i, 0)),
            pl.BlockSpec((RB, 32), lambda i: (i, 0)),
            pl.BlockSpec((RB, 32), lambda i: (i, 0)),
        ],
        out_shape=[
            jax.ShapeDtypeStruct((N, 1), jnp.float32),
            jax.ShapeDtypeStruct((N, 32), jnp.float32),
            jax.ShapeDtypeStruct((N, 32), jnp.float32),
        ],
    )(x, h, c, t2, t2, wcat, w0, w1a, w1b, bcat, wci, wcf, wco, lw, lb)


def kernel(x, edge_index, edge_weight, h, c, params):
    p = params
    ei3 = edge_index.reshape(2, EG, G)
    w2 = edge_weight.reshape(EG, G)
    h2 = h.reshape(2 * N, 16)

    deg_parts = _deg_call(ei3, w2)
    dis_g = _dis_call(deg_parts.reshape(2, 782, 128))
    dis_flat = dis_g.reshape(NP)
    t2 = _prop_call(ei3, w2, h2, dis_flat)

    gates = ["i", "f", "c", "o"]
    wcat = jnp.concatenate([p["W_" + g] for g in gates], axis=1)
    w0 = jnp.concatenate([p["conv_" + g + "_w"][0] for g in gates], axis=1)
    w1 = jnp.concatenate([p["conv_" + g + "_w"][1] for g in gates], axis=1)
    bcat = jnp.concatenate(
        [p["conv_" + g + "_b"][None, :] + p["b_" + g] for g in gates], axis=1
    )
    y, hn, cn = _dense_call(
        x, h, c, t2,
        wcat, w0, w1[:16], w1[16:], bcat,
        p["w_c_i"], p["w_c_f"], p["w_c_o"],
        p["lin_w"], p["lin_b"].reshape(1, 1),
    )
    return (y, hn, cn)


# speedup vs baseline: 43.3139x; 1.2391x over previous
import jax
import jax.numpy as jnp
from jax import lax
from jax.experimental import pallas as pl
from jax.experimental.pallas import tpu as pltpu
from jax.experimental.pallas import tpu_sc as plsc

N = 100000
NP = 100096
NSLICE = 6256
E = 1600000
G = 128
EG = E // G
CH = 8
PC = 4
RB = 4352

_SC_PARAMS = pltpu.CompilerParams(use_tc_tiling_on_sc=False)


def _sc_mesh():
    return plsc.VectorSubcoreMesh(
        core_axis_name="c", subcore_axis_name="s", num_cores=2, num_subcores=16
    )


def _deg_body(ei_hbm, w_hbm, out_hbm, rowb, wb, stage, acc, sem):
    cid = lax.axis_index("c")
    sid = lax.axis_index("s")

    def zbody(i, carry):
        stage[pl.ds(i * 16, 16)] = jnp.zeros((16,), jnp.float32)
        return carry

    lax.fori_loop(0, NSLICE // 16, zbody, 0)
    pltpu.sync_copy(stage, acc.at[pl.ds(sid * NSLICE, NSLICE)])
    plsc.subcore_barrier()

    wkr = cid * 16 + sid
    g0 = wkr * 390 + jnp.minimum(wkr, 20)
    rem = jnp.where(wkr < 20, 7, 6)

    def chunk(ci, carry):
        gr = g0 + ci * CH
        ld = [
            pltpu.async_copy(ei_hbm.at[0, pl.ds(gr, CH)], rowb, sem),
            pltpu.async_copy(w_hbm.at[pl.ds(gr, CH)], wb, sem),
        ]
        for cp in ld:
            cp.wait()
        cps = [
            pltpu.async_copy(wb.at[j], acc.at[rowb.at[j]], sem, add=True)
            for j in range(CH)
        ]
        for cp in cps:
            cp.wait()
        return carry

    lax.fori_loop(0, 48, chunk, 0)

    def tail(t, carry):
        gr = g0 + 384 + t
        ld = [
            pltpu.async_copy(ei_hbm.at[0, gr], rowb.at[0], sem),
            pltpu.async_copy(w_hbm.at[gr], wb.at[0], sem),
        ]
        for cp in ld:
            cp.wait()
        pltpu.async_copy(wb.at[0], acc.at[rowb.at[0]], sem, add=True).wait()
        return carry

    lax.fori_loop(0, rem, tail, 0)
    plsc.subcore_barrier()
    pltpu.sync_copy(acc.at[pl.ds(sid * NSLICE, NSLICE)], stage)
    pltpu.sync_copy(stage, out_hbm.at[pl.ds(cid * NP + sid * NSLICE, NSLICE)])


def _deg_call(ei3, w2):
    return pl.kernel(
        _deg_body,
        out_type=jax.ShapeDtypeStruct((2 * NP,), jnp.float32),
        mesh=_sc_mesh(),
        compiler_params=_SC_PARAMS,
        scratch_types=[
            pltpu.VMEM((CH, G), jnp.int32),
            pltpu.VMEM((CH, G), jnp.float32),
            pltpu.VMEM((NSLICE,), jnp.float32),
            pltpu.VMEM_SHARED((NP,), jnp.float32),
            pltpu.SemaphoreType.DMA,
        ],
    )(ei3, w2)


def _dis_body(p0_ref, p1_ref, dis_ref):
    deg = p0_ref[0] + p1_ref[0]
    dis_ref[...] = jnp.where(deg > 0.0, lax.rsqrt(deg), 0.0)


def _dis_call(parts3):
    return pl.pallas_call(
        _dis_body,
        grid=(1,),
        in_specs=[
            pl.BlockSpec((1, 782, 128), lambda i: (0, 0, 0)),
            pl.BlockSpec((1, 782, 128), lambda i: (1, 0, 0)),
        ],
        out_specs=pl.BlockSpec((782, 128), lambda i: (0, 0)),
        out_shape=jax.ShapeDtypeStruct((782, 128), jnp.float32),
    )(parts3, parts3)


def _scale_rows(rows, scb, nsub16, base, negate=False):
    def sbody(i, carry):
        sv = scb[pl.ds(i * 16, 16)]
        if negate:
            sv = -sv
        e0 = base + i * 16
        for u in range(16):
            rows[e0 + u, :] = rows[e0 + u, :] * sv[u]
        return carry

    lax.fori_loop(0, nsub16, sbody, 0)


def _prop_body(ei_hbm, w_hbm, h2_hbm, dis_hbm, out_hbm,
               rowb0, rowb1, colb0, colb1, csb0, csb1, wbf, disrb,
               scb0, scb1, rows0, rows1, disv, acc,
               semi, semg0, semg1, sems0, sems1):
    cid = lax.axis_index("c")
    sid = lax.axis_index("s")
    rowb = (rowb0, rowb1)
    colb = (colb0, colb1)
    csb = (csb0, csb1)
    scb = (scb0, scb1)
    rows = (rows0, rows1)
    semg = (semg0, semg1)
    sems = (sems0, sems1)

    def zbody(i, carry):
        rows0[i, :] = jnp.zeros((16,), jnp.float32)
        return carry

    lax.fori_loop(0, PC * G, zbody, 0)
    nz = PC * G
    for k in range(NSLICE // nz):
        pltpu.sync_copy(rows0, acc.at[pl.ds(sid * NSLICE + k * nz, nz)])
    rz = NSLICE - (NSLICE // nz) * nz
    pltpu.sync_copy(
        rows0.at[pl.ds(0, rz)],
        acc.at[pl.ds(sid * NSLICE + (NSLICE // nz) * nz, rz)],
    )
    plsc.subcore_barrier()

    g0 = sid * 781 + jnp.minimum(sid, 4)
    rem = jnp.where(sid < 4, 2, 1)

    def vloop(n, body):
        lax.fori_loop(0, n, lambda i, c: (body(i), c)[1], 0)

    def drain(sem_, buf):
        pltpu.make_async_copy(h2_hbm.at[pl.ds(0, PC * G)], buf, sem_).wait()

    def stage_a(ci, s, drain_sc):
        gr = g0 + ci * PC
        ld = [
            pltpu.async_copy(ei_hbm.at[0, pl.ds(gr, PC)], rowb[s], semi),
            pltpu.async_copy(ei_hbm.at[1, pl.ds(gr, PC)], colb[s], semi),
            pltpu.async_copy(w_hbm.at[pl.ds(gr, PC)], wbf, semi),
        ]
        for cp in ld:
            cp.wait()
        gd = [
            pltpu.async_copy(dis_hbm.at[rowb[s].at[j]], disrb.at[j], semi)
            for j in range(PC)
        ]
        for cp in gd:
            cp.wait()
        if drain_sc:
            drain(sems[s], rows[s])

        def obody(i):
            j = i // (G // 16)
            l = i % (G // 16)
            sl = (j, pl.ds(l * 16, 16))
            rowb[s][sl] = rowb[s][sl] * 2 + cid
            scb[s][sl] = wbf[sl] * disrb[sl]
            csb[s][sl] = colb[s][sl]

        vloop(PC * (G // 16), obody)
        for j in range(PC):
            pltpu.async_copy(
                h2_hbm.at[rowb[s].at[j]], rows[s].at[pl.ds(j * G, G)], semg[s]
            )

    def stage_b(s):
        drain(semg[s], rows[s])

        def sbody(i):
            j = i // (G // 16)
            l = i % (G // 16)
            sv = scb[s][j, pl.ds(l * 16, 16)]
            e0 = i * 16
            for u in range(16):
                rows[s][e0 + u, :] = rows[s][e0 + u, :] * sv[u]

        vloop(PC * (G // 16), sbody)
        for j in range(PC):
            pltpu.async_copy(
                rows[s].at[pl.ds(j * G, G)], acc.at[csb[s].at[j]],
                sems[s], add=True,
            )

    stage_a(0, 0, False)
    stage_a(1, 1, False)
    stage_b(0)

    def ring(k, carry):
        ci = 2 * k
        stage_a(ci, 0, True)
        stage_b(1)
        stage_a(ci + 1, 1, True)
        stage_b(0)
        return carry

    lax.fori_loop(1, 97, ring, 0)
    stage_a(194, 0, True)
    stage_b(1)
    stage_b(0)
    drain(sems[0], rows[0])
    drain(sems[1], rows[1])

    def tail(t, carry):
        gr = g0 + 780 + t
        ld = [
            pltpu.async_copy(ei_hbm.at[0, gr], rowb0.at[0], semi),
            pltpu.async_copy(ei_hbm.at[1, gr], colb0.at[0], semi),
            pltpu.async_copy(w_hbm.at[gr], wbf.at[0], semi),
        ]
        for cp in ld:
            cp.wait()
        pltpu.async_copy(dis_hbm.at[rowb0.at[0]], disrb.at[0], semi).wait()

        def obody(i):
            sl = (0, pl.ds(i * 16, 16))
            rowb0[sl] = rowb0[sl] * 2 + cid
            scb0[sl] = wbf[sl] * disrb[sl]

        vloop(G // 16, obody)
        pltpu.async_copy(
            h2_hbm.at[rowb0.at[0]], rows0.at[pl.ds(0, G)], semi
        ).wait()

        def sbody(i):
            sv = scb0[0, pl.ds(i * 16, 16)]
            e0 = i * 16
            for u in range(16):
                rows0[e0 + u, :] = rows0[e0 + u, :] * sv[u]

        vloop(G // 16, sbody)
        pltpu.async_copy(
            rows0.at[pl.ds(0, G)], acc.at[colb0.at[0]], semi, add=True
        ).wait()
        return carry

    lax.fori_loop(0, rem, tail, 0)
    plsc.subcore_barrier()

    node0 = sid * NSLICE
    pltpu.sync_copy(dis_hbm.at[pl.ds(node0, NSLICE)], disv)
    nwb = PC * G
    for k in range(13):
        cnt = nwb if k < 12 else NSLICE - 12 * nwb
        pltpu.sync_copy(
            acc.at[pl.ds(node0 + k * nwb, cnt)], rows0.at[pl.ds(0, cnt)]
        )

        def wbody(i, carry, k=k):
            sv = -disv[pl.ds(k * nwb + i * 16, 16)]
            e0 = i * 16
            for u in range(16):
                rows0[e0 + u, :] = rows0[e0 + u, :] * sv[u]
            return carry

        lax.fori_loop(0, cnt // 16, wbody, 0)
        pltpu.sync_copy(
            rows0.at[pl.ds(0, cnt)],
            out_hbm.at[pl.ds(cid * NP + node0 + k * nwb, cnt)],
        )


def _prop_call(ei3, w2, h2, dis_flat):
    return pl.kernel(
        _prop_body,
        out_type=jax.ShapeDtypeStruct((2 * NP, 16), jnp.float32),
        mesh=_sc_mesh(),
        compiler_params=_SC_PARAMS,
        scratch_types=[
            pltpu.VMEM((PC, G), jnp.int32),
            pltpu.VMEM((PC, G), jnp.int32),
            pltpu.VMEM((PC, G), jnp.int32),
            pltpu.VMEM((PC, G), jnp.int32),
            pltpu.VMEM((PC, G), jnp.int32),
            pltpu.VMEM((PC, G), jnp.int32),
            pltpu.VMEM((PC, G), jnp.float32),
            pltpu.VMEM((PC, G), jnp.float32),
            pltpu.VMEM((PC, G), jnp.float32),
            pltpu.VMEM((PC, G), jnp.float32),
            pltpu.VMEM((PC * G, 16), jnp.float32),
            pltpu.VMEM((PC * G, 16), jnp.float32),
            pltpu.VMEM((NSLICE,), jnp.float32),
            pltpu.VMEM_SHARED((NP, 16), jnp.float32),
            pltpu.SemaphoreType.DMA,
            pltpu.SemaphoreType.DMA,
            pltpu.SemaphoreType.DMA,
            pltpu.SemaphoreType.DMA,
            pltpu.SemaphoreType.DMA,
        ],
    )(ei3, w2, h2, dis_flat)


def _dgt(w, v):
    cdim = 0 if v.shape[0] == w.shape[0] else 1
    return lax.dot_general(
        w, v, (((0,), (cdim,)), ((), ())),
        preferred_element_type=jnp.float32,
    )


def _dense_body(xt_ref, ht_ref, ct_ref, t0_ref, t1_ref,
                wcat_ref, w0_ref, w1a_ref, w1b_ref, bcat_ref,
                wci_ref, wcf_ref, wco_ref, lw_ref, lb_ref,
                y_ref, hh_ref, cc_ref):
    s = (
        _dgt(wcat_ref[...], xt_ref[...])
        + _dgt(w0_ref[...], ht_ref[...])
        + _dgt(w1a_ref[...], t0_ref[...])
        + _dgt(w1b_ref[...], t1_ref[...])
        + bcat_ref[...]
    )
    cold = ct_ref[...]
    gi = jax.nn.sigmoid(s[0:32, :] + wci_ref[...] * cold)
    gf = jax.nn.sigmoid(s[32:64, :] + wcf_ref[...] * cold)
    gt = jnp.tanh(s[64:96, :])
    cnew = gf * cold + gi * gt
    go = jax.nn.sigmoid(s[96:128, :] + wco_ref[...] * cnew)
    hnew = go * jnp.tanh(cnew)
    cc_ref[...] = cnew
    hh_ref[...] = hnew
    y_ref[...] = _dgt(lw_ref[...], jax.nn.relu(hnew)) + lb_ref[...]


def _dense_call(xt, ht, ct, t2, wcat, w0, w1a, w1b, bcatT,
                wciT, wcfT, wcoT, lw, lb):
    full = lambda shape: pl.BlockSpec(shape, lambda i: tuple(0 for _ in shape))
    nblk = (N + RB - 1) // RB
    return pl.pallas_call(
        _dense_body,
        grid=(nblk,),
        in_specs=[
            pl.BlockSpec((8, RB), lambda i: (0, i)),
            pl.BlockSpec((32, RB), lambda i: (0, i)),
            pl.BlockSpec((32, RB), lambda i: (0, i)),
            pl.BlockSpec((RB, 16), lambda i: (i, 0)),
            pl.BlockSpec((RB, 16), lambda i: (23 + i, 0)),
            full((8, 128)),
            full((32, 128)),
            full((16, 128)),
            full((16, 128)),
            full((128, 1)),
            full((32, 1)),
            full((32, 1)),
            full((32, 1)),
            full((32, 1)),
            full((1, 1)),
        ],
        out_specs=[
            pl.BlockSpec((1, RB), lambda i: (0, i)),
            pl.BlockSpec((32, RB), lambda i: (0, i)),
            pl.BlockSpec((32, RB), lambda i: (0, i)),
        ],
        out_shape=[
            jax.ShapeDtypeStruct((1, N), jnp.float32),
            jax.ShapeDtypeStruct((32, N), jnp.float32),
            jax.ShapeDtypeStruct((32, N), jnp.float32),
        ],
    )(xt, ht, ct, t2, t2, wcat, w0, w1a, w1b, bcatT,
      wciT, wcfT, wcoT, lw, lb)


def kernel(x, edge_index, edge_weight, h, c, params):
    p = params
    ei3 = edge_index.reshape(2, EG, G)
    w2 = edge_weight.reshape(EG, G)
    h2 = h.reshape(2 * N, 16)

    deg_parts = _deg_call(ei3, w2)
    dis_g = _dis_call(deg_parts.reshape(2, 782, 128))
    dis_flat = dis_g.reshape(NP)
    t2 = _prop_call(ei3, w2, h2, dis_flat)

    gates = ["i", "f", "c", "o"]
    wcat = jnp.concatenate([p["W_" + g] for g in gates], axis=1)
    w0 = jnp.concatenate([p["conv_" + g + "_w"][0] for g in gates], axis=1)
    w1 = jnp.concatenate([p["conv_" + g + "_w"][1] for g in gates], axis=1)
    bcat = jnp.concatenate(
        [p["conv_" + g + "_b"][None, :] + p["b_" + g] for g in gates], axis=1
    )
    yt, hnt, cnt = _dense_call(
        x.T, h.T, c.T, t2,
        wcat, w0, w1[:16], w1[16:], bcat.T,
        p["w_c_i"].T, p["w_c_f"].T, p["w_c_o"].T,
        p["lin_w"], p["lin_b"].reshape(1, 1),
    )
    return (yt.reshape(N, 1), hnt.T, cnt.T)
